# baseline (speedup 1.0000x reference)
import functools

import jax
import jax.numpy as jnp
from jax import lax
from jax.experimental import pallas as pl
from jax.experimental.pallas import tpu as pltpu
from jax.experimental.pallas import tpu_sc as plsc

B = 16384
F = 26
V = 100000
D = 16
H0, H1, H2 = 256, 128, 64
FD = F * D

NC, NS = 2, 16
NW = NC * NS
N = B * F

F0 = 16
F1 = F - F0

VB = 8192
NVB = 13
VPG = NVB * VB
NGH = 2
NROW16_H = NGH * VPG * 8


def _flatten_body(embT_ref, out_ref):
    out_ref[...] = embT_ref[...].T


def _tc_flatten(embT2_half):
    return pl.pallas_call(
        _flatten_body,
        grid=(NGH, NVB),
        in_specs=[pl.BlockSpec((128, VB), lambda g, j: (g, j))],
        out_specs=pl.BlockSpec((VB, 128), lambda g, j: (g * NVB + j, 0)),
        out_shape=jax.ShapeDtypeStruct((NGH * VPG, 128), jnp.float32),
    )(embT2_half)


def _sc_gather(idx_e, emb_flat, per_w, ch, idx_l=None, lin_flat=None):
    mesh = plsc.VectorSubcoreMesh(core_axis_name="c", subcore_axis_name="s")
    n = idx_e.shape[0]
    nch = per_w // ch
    with_lin = idx_l is not None
    per_w_l = N // NW

    out_type = [jax.ShapeDtypeStruct((n, D), jnp.float32)]
    scratch = [
        pltpu.VMEM((per_w,), jnp.int32),
        pltpu.VMEM((2, ch, D), jnp.float32),
        pltpu.SemaphoreType.DMA,
        pltpu.SemaphoreType.DMA,
    ]
    if with_lin:
        out_type.append(jax.ShapeDtypeStruct((N,), jnp.float32))
        scratch += [
            pltpu.VMEM((per_w_l,), jnp.int32),
            pltpu.VMEM((per_w_l,), jnp.float32),
            pltpu.SemaphoreType.DMA,
        ]

    @functools.partial(
        pl.kernel,
        out_type=tuple(out_type),
        mesh=mesh,
        compiler_params=pltpu.CompilerParams(use_tc_tiling_on_sc=False),
        scratch_types=scratch,
    )
    def k(*refs):
        if with_lin:
            (idxe_hbm, idxl_hbm, emb_hbm, lin_hbm, rows_out, lin_out,
             idxe_v, rows_v, sem0, sem1, idxl_v, lin_v, seml) = refs
        else:
            (idxe_hbm, emb_hbm, rows_out,
             idxe_v, rows_v, sem0, sem1) = refs
        wid = lax.axis_index("s") * NC + lax.axis_index("c")
        base = wid * per_w
        pltpu.sync_copy(idxe_hbm.at[pl.ds(base, per_w)], idxe_v)
        if with_lin:
            basel = wid * per_w_l
            pltpu.sync_copy(idxl_hbm.at[pl.ds(basel, per_w_l)], idxl_v)
            lin_cp = pltpu.async_copy(lin_hbm.at[idxl_v], lin_v, seml)
        sems = (sem0, sem1)
        cps = [None, None]
        cps[0] = pltpu.async_copy(
            emb_hbm.at[idxe_v.at[pl.ds(0, ch)]], rows_v.at[0], sem0)
        for c in range(1, nch + 1):
            if c < nch:
                cps[c % 2] = pltpu.async_copy(
                    emb_hbm.at[idxe_v.at[pl.ds(c * ch, ch)]],
                    rows_v.at[c % 2], sems[c % 2])
            cps[(c - 1) % 2].wait()
            pltpu.sync_copy(rows_v.at[(c - 1) % 2],
                            rows_out.at[pl.ds(base + (c - 1) * ch, ch)])
        if with_lin:
            lin_cp.wait()
            pltpu.sync_copy(lin_v, lin_out.at[pl.ds(basel, per_w_l)])

    if with_lin:
        return k(idx_e, idx_l, emb_flat, lin_flat)
    return k(idx_e, emb_flat)


BM = 1024


def _mlp_body(feat, linv, bias, w0, b0, w1, b1, w2, b2, w3, b3, out):
    x = feat[...]
    h = jnp.maximum(jnp.dot(x, w0[...], preferred_element_type=jnp.float32)
                    + b0[...], 0.0)
    h = jnp.maximum(jnp.dot(h, w1[...], preferred_element_type=jnp.float32)
                    + b1[...], 0.0)
    h = jnp.maximum(jnp.dot(h, w2[...], preferred_element_type=jnp.float32)
                    + b2[...], 0.0)
    o = jnp.dot(h, w3[...], preferred_element_type=jnp.float32) + b3[...]
    wide = jnp.sum(linv[...], axis=1, keepdims=True) + bias[...]
    out[...] = jax.nn.sigmoid(o + wide)


def _tc_mlp(feat, linv, bias, W0, b0, W1, b1, W2, b2, W3, b3):
    grid = (B // BM,)
    const = lambda i: (0, 0)
    return pl.pallas_call(
        _mlp_body,
        grid=grid,
        in_specs=[
            pl.BlockSpec((BM, FD), lambda i: (i, 0)),
            pl.BlockSpec((BM, F), lambda i: (i, 0)),
            pl.BlockSpec((1, 1), const),
            pl.BlockSpec((FD, H0), const),
            pl.BlockSpec((1, H0), const),
            pl.BlockSpec((H0, H1), const),
            pl.BlockSpec((1, H1), const),
            pl.BlockSpec((H1, H2), const),
            pl.BlockSpec((1, H2), const),
            pl.BlockSpec((H2, 1), const),
            pl.BlockSpec((1, 1), const),
        ],
        out_specs=pl.BlockSpec((BM, 1), lambda i: (i, 0)),
        out_shape=jax.ShapeDtypeStruct((B, 1), jnp.float32),
    )(feat, linv, bias, W0, b0, W1, b1, W2, b2, W3, b3)


def _emb_idx(xi_cols, f_local):
    g = f_local // 8
    return (((g * NVB)[None, :] + xi_cols // VB) * (VB * 8)
            + (xi_cols % VB) * 8 + (f_local % 8)[None, :])


def kernel(x, lin_tables, emb_tables, bias, W0, b0, W1, b1, W2, b2, W3, b3):
    xi = x.astype(jnp.int32)
    f_rng = jnp.arange(F, dtype=jnp.int32)
    idx_e0 = _emb_idx(xi[:, :F0], f_rng[:F0]).reshape(B * F0)
    idx_e1 = _emb_idx(xi[:, F0:], f_rng[F0:] - F0).reshape(B * F1)
    idx_l = (xi + (f_rng * V)[None, :]).reshape(N)
    embT2 = jnp.transpose(emb_tables, (0, 2, 1)).reshape(F * D, V)
    lin_flat = lin_tables.reshape(F * V)
    flat0 = _tc_flatten(embT2[:F0 * D]).reshape(NROW16_H, D)
    rows0, linv = _sc_gather(idx_e0, flat0, per_w=B * F0 // NW, ch=2048,
                             idx_l=idx_l, lin_flat=lin_flat)
    flat1 = _tc_flatten(embT2[F0 * D:]).reshape(NROW16_H, D)
    rows1 = _sc_gather(idx_e1, flat1, per_w=B * F1 // NW, ch=2560)[0]
    feat = jnp.concatenate(
        [rows0.reshape(B, F0 * D), rows1.reshape(B, F1 * D)], axis=1)
    linv = linv.reshape(B, F)
    out = _tc_mlp(feat, linv, bias.reshape(1, 1), W0, b0.reshape(1, H0),
                  W1, b1.reshape(1, H1), W2, b2.reshape(1, H2),
                  W3, b3.reshape(1, 1))
    return out.reshape(B)


# speedup vs baseline: 1.3929x; 1.3929x over previous
import functools

import jax
import jax.numpy as jnp
from jax import lax
from jax.experimental import pallas as pl
from jax.experimental.pallas import tpu as pltpu
from jax.experimental.pallas import tpu_sc as plsc

B = 16384
F = 26
V = 100000
D = 16
H0, H1, H2 = 256, 128, 64
FD = F * D

NC, NS = 2, 16
NW = NC * NS
N = B * F

VB = 8192
NVB = 13
VPG = NVB * VB


def _flatten_body(embT_ref, out_ref):
    out_ref[...] = embT_ref[...].T


def _tc_flatten(embT2):
    ng = -(-embT2.shape[0] // 128)
    return pl.pallas_call(
        _flatten_body,
        grid=(ng, NVB),
        in_specs=[pl.BlockSpec((128, VB), lambda g, j: (g, j))],
        out_specs=pl.BlockSpec((VB, 128), lambda g, j: (g * NVB + j, 0)),
        out_shape=jax.ShapeDtypeStruct((ng * VPG, 128), jnp.float32),
    )(embT2)


def _sc_gather(idx_e, emb_flat, per_w, ch, idx_l=None, lin_flat=None):
    mesh = plsc.VectorSubcoreMesh(core_axis_name="c", subcore_axis_name="s")
    n = idx_e.shape[0]
    nch = per_w // ch
    with_lin = idx_l is not None
    per_w_l = N // NW

    out_type = [jax.ShapeDtypeStruct((n, D), jnp.float32)]
    scratch = [
        pltpu.VMEM((per_w,), jnp.int32),
        pltpu.VMEM((2, ch, D), jnp.float32),
        pltpu.SemaphoreType.DMA,
        pltpu.SemaphoreType.DMA,
    ]
    if with_lin:
        out_type.append(jax.ShapeDtypeStruct((N,), jnp.float32))
        scratch += [
            pltpu.VMEM((per_w_l,), jnp.int32),
            pltpu.VMEM((per_w_l,), jnp.float32),
            pltpu.SemaphoreType.DMA,
        ]

    @functools.partial(
        pl.kernel,
        out_type=tuple(out_type),
        mesh=mesh,
        compiler_params=pltpu.CompilerParams(use_tc_tiling_on_sc=False),
        scratch_types=scratch,
    )
    def k(*refs):
        if with_lin:
            (idxe_hbm, idxl_hbm, emb_hbm, lin_hbm, rows_out, lin_out,
             idxe_v, rows_v, sem0, sem1, idxl_v, lin_v, seml) = refs
        else:
            (idxe_hbm, emb_hbm, rows_out,
             idxe_v, rows_v, sem0, sem1) = refs
        wid = lax.axis_index("s") * NC + lax.axis_index("c")
        base = wid * per_w
        pltpu.sync_copy(idxe_hbm.at[pl.ds(base, per_w)], idxe_v)
        if with_lin:
            basel = wid * per_w_l
            pltpu.sync_copy(idxl_hbm.at[pl.ds(basel, per_w_l)], idxl_v)
            lin_cp = pltpu.async_copy(lin_hbm.at[idxl_v], lin_v, seml)
        sems = (sem0, sem1)
        cps = [None, None]
        cps[0] = pltpu.async_copy(
            emb_hbm.at[idxe_v.at[pl.ds(0, ch)]], rows_v.at[0], sem0)
        for c in range(1, nch + 1):
            if c < nch:
                cps[c % 2] = pltpu.async_copy(
                    emb_hbm.at[idxe_v.at[pl.ds(c * ch, ch)]],
                    rows_v.at[c % 2], sems[c % 2])
            cps[(c - 1) % 2].wait()
            pltpu.sync_copy(rows_v.at[(c - 1) % 2],
                            rows_out.at[pl.ds(base + (c - 1) * ch, ch)])
        if with_lin:
            lin_cp.wait()
            pltpu.sync_copy(lin_v, lin_out.at[pl.ds(basel, per_w_l)])

    if with_lin:
        return k(idx_e, idx_l, emb_flat, lin_flat)
    return k(idx_e, emb_flat)


BM = 1024


def _mlp_body(feat, linv, bias, w0, b0, w1, b1, w2, b2, w3, b3, out):
    x = feat[...]
    h = jnp.maximum(jnp.dot(x, w0[...], preferred_element_type=jnp.float32)
                    + b0[...], 0.0)
    h = jnp.maximum(jnp.dot(h, w1[...], preferred_element_type=jnp.float32)
                    + b1[...], 0.0)
    h = jnp.maximum(jnp.dot(h, w2[...], preferred_element_type=jnp.float32)
                    + b2[...], 0.0)
    o = jnp.dot(h, w3[...], preferred_element_type=jnp.float32) + b3[...]
    wide = jnp.sum(linv[...], axis=1, keepdims=True) + bias[...]
    out[...] = jax.nn.sigmoid(o + wide)


def _tc_mlp(feat, linv, bias, W0, b0, W1, b1, W2, b2, W3, b3):
    grid = (B // BM,)
    const = lambda i: (0, 0)
    return pl.pallas_call(
        _mlp_body,
        grid=grid,
        in_specs=[
            pl.BlockSpec((BM, FD), lambda i: (i, 0)),
            pl.BlockSpec((BM, F), lambda i: (i, 0)),
            pl.BlockSpec((1, 1), const),
            pl.BlockSpec((FD, H0), const),
            pl.BlockSpec((1, H0), const),
            pl.BlockSpec((H0, H1), const),
            pl.BlockSpec((1, H1), const),
            pl.BlockSpec((H1, H2), const),
            pl.BlockSpec((1, H2), const),
            pl.BlockSpec((H2, 1), const),
            pl.BlockSpec((1, 1), const),
        ],
        out_specs=pl.BlockSpec((BM, 1), lambda i: (i, 0)),
        out_shape=jax.ShapeDtypeStruct((B, 1), jnp.float32),
    )(feat, linv, bias, W0, b0, W1, b1, W2, b2, W3, b3)


def _emb_idx(xi_cols, f_local):
    g = f_local // 8
    return (((g * NVB)[None, :] + xi_cols // VB) * (VB * 8)
            + (xi_cols % VB) * 8 + (f_local % 8)[None, :])


def kernel(x, lin_tables, emb_tables, bias, W0, b0, W1, b1, W2, b2, W3, b3):
    xi = x.astype(jnp.int32)
    f_rng = jnp.arange(F, dtype=jnp.int32)
    idx_e = _emb_idx(xi, f_rng).reshape(N)
    idx_l = (xi + (f_rng * V)[None, :]).reshape(N)
    embT2 = jnp.transpose(emb_tables, (0, 2, 1)).reshape(F * D, V)
    lin_flat = lin_tables.reshape(F * V)
    flat = _tc_flatten(embT2).reshape(4 * VPG * 8, D)
    rows, linv = _sc_gather(idx_e, flat, per_w=N // NW, ch=1664,
                            idx_l=idx_l, lin_flat=lin_flat)
    feat = rows.reshape(B, FD)
    linv = linv.reshape(B, F)
    out = _tc_mlp(feat, linv, bias.reshape(1, 1), W0, b0.reshape(1, H0),
                  W1, b1.reshape(1, H1), W2, b2.reshape(1, H2),
                  W3, b3.reshape(1, 1))
    return out.reshape(B)
